# baseline (speedup 1.0000x reference)
import jax
import jax.numpy as jnp
from jax import lax
from jax.experimental import pallas as pl
from jax.experimental.pallas import tpu as pltpu
from jax.experimental.pallas import tpu_sc as plsc

_B, _H, _S, _D = 16, 16, 4096, 64
_L = 16
_BH = _B * _H
_NW = 32
_SL = _BH // _NW
_CHS = 1024
_NCH = _S // _CHS


def _sc_body(kc, vc, pos, alive, kval, vval, ko, vo, idx_v, alive_v, kvb, vvb, buf):
    c = lax.axis_index("c")
    s = lax.axis_index("s")
    wid = s * 2 + c
    base = wid * _SL
    pltpu.sync_copy(pos, idx_v)
    pltpu.sync_copy(alive, alive_v)
    lane = lax.iota(jnp.int32, 16)

    for src, vsrc, vbuf, dst in ((kc, kval, kvb, ko), (vc, vval, vvb, vo)):
        def slice_body(i, _, src=src, vsrc=vsrc, vbuf=vbuf, dst=dst):
            bh = base + i
            pltpu.sync_copy(vsrc.at[bh], vbuf)

            def chunk_body(jc, _):
                off = jc * _CHS
                pltpu.sync_copy(src.at[bh, :, pl.ds(off, _CHS)], buf)

                def pos_body(l, _):
                    lsplat = jnp.full((16,), 0, jnp.int32) + l
                    psp = plsc.load_gather(idx_v, [lsplat])
                    asp = plsc.load_gather(alive_v, [lsplat])
                    rel = psp - off
                    m = (rel >= 0) & (rel < _CHS) & (asp != 0)
                    relc = jnp.clip(rel, 0, _CHS - 1)
                    for g in range(4):
                        rows = g * 16 + lane
                        col = plsc.load_gather(vbuf, [rows, lsplat])
                        plsc.store_scatter(buf, [rows, relc], col, mask=m)
                    return 0

                lax.fori_loop(0, _L, pos_body, 0)
                pltpu.sync_copy(buf, dst.at[bh, :, pl.ds(off, _CHS)])
                return 0

            lax.fori_loop(0, _NCH, chunk_body, 0)
            return 0

        lax.fori_loop(0, _SL, slice_body, 0)


def kernel(k_cache, v_cache, input_pos, k_val, v_val, interpret=False):
    kct = jnp.swapaxes(k_cache, 2, 3).reshape(_BH, _D, _S)
    vct = jnp.swapaxes(v_cache, 2, 3).reshape(_BH, _D, _S)
    kvt = jnp.swapaxes(k_val, 2, 3).reshape(_BH, _D, _L)
    vvt = jnp.swapaxes(v_val, 2, 3).reshape(_BH, _D, _L)
    nxt = jnp.concatenate([input_pos[1:], jnp.full((1,), -1, jnp.int32)])
    alive = (input_pos != nxt).astype(jnp.int32)

    mesh = plsc.VectorSubcoreMesh(core_axis_name="c", subcore_axis_name="s")
    f = pl.kernel(
        _sc_body,
        out_type=[jax.ShapeDtypeStruct((_BH, _D, _S), jnp.float32)] * 2,
        mesh=mesh,
        scratch_types=[
            pltpu.VMEM((_L,), jnp.int32),
            pltpu.VMEM((_L,), jnp.int32),
            pltpu.VMEM((_D, _L), jnp.float32),
            pltpu.VMEM((_D, _L), jnp.float32),
            pltpu.VMEM((_D, _CHS), jnp.float32),
        ],
        compiler_params=pltpu.CompilerParams(needs_layout_passes=False),
        interpret=interpret,
    )
    ko, vo = f(kct, vct, input_pos, alive, kvt, vvt)
    ko = jnp.swapaxes(ko.reshape(_B, _H, _D, _S), 2, 3)
    vo = jnp.swapaxes(vo.reshape(_B, _H, _D, _S), 2, 3)
    return ko, vo


# speedup vs baseline: 1.1981x; 1.1981x over previous
import jax
import jax.numpy as jnp
from jax import lax
from jax.experimental import pallas as pl
from jax.experimental.pallas import tpu as pltpu
from jax.experimental.pallas import tpu_sc as plsc

_B, _H, _S, _D = 16, 16, 4096, 64
_L = 16
_BH = _B * _H
_G = 2
_NW = 32
_SL = _BH // _NW
_CHS = 1024
_NCH = _S // _CHS


def _tc_body(kc, kv, oh, cm, ko):
    mask = cm[...] > 0
    for g in range(_G):
        dk = jax.lax.dot(
            kv[g], oh[...], precision=jax.lax.Precision.HIGHEST,
            preferred_element_type=jnp.float32,
        )
        ko[g] = jnp.where(mask, dk, kc[g])


def _sc_body(vc, pos, alive, vval, vo, idx_v, alive_v, vvb, buf):
    c = lax.axis_index("c")
    s = lax.axis_index("s")
    wid = s * 2 + c
    base = wid * _SL
    pltpu.sync_copy(pos, idx_v)
    pltpu.sync_copy(alive, alive_v)
    lane = lax.iota(jnp.int32, 16)

    def slice_body(i, _):
        bh = base + i
        pltpu.sync_copy(vval.at[bh], vvb)

        def chunk_body(jc, _):
            off = jc * _CHS
            pltpu.sync_copy(vc.at[bh, :, pl.ds(off, _CHS)], buf)

            def pos_body(l, _):
                lsplat = jnp.full((16,), 0, jnp.int32) + l
                psp = plsc.load_gather(idx_v, [lsplat])
                asp = plsc.load_gather(alive_v, [lsplat])
                rel = psp - off
                m = (rel >= 0) & (rel < _CHS) & (asp != 0)
                relc = jnp.clip(rel, 0, _CHS - 1)
                for g in range(4):
                    rows = g * 16 + lane
                    col = plsc.load_gather(vvb, [rows, lsplat])
                    plsc.store_scatter(buf, [rows, relc], col, mask=m)
                return 0

            lax.fori_loop(0, _L, pos_body, 0)
            pltpu.sync_copy(buf, vo.at[bh, :, pl.ds(off, _CHS)])
            return 0

        lax.fori_loop(0, _NCH, chunk_body, 0)
        return 0

    lax.fori_loop(0, _SL, slice_body, 0)


def kernel(k_cache, v_cache, input_pos, k_val, v_val):
    kct = jnp.swapaxes(k_cache, 2, 3).reshape(_BH, _D, _S)
    vct = jnp.swapaxes(v_cache, 2, 3).reshape(_BH, _D, _S)
    kvt = jnp.swapaxes(k_val, 2, 3).reshape(_BH, _D, _L)
    vvt = jnp.swapaxes(v_val, 2, 3).reshape(_BH, _D, _L)

    nxt = jnp.concatenate([input_pos[1:], jnp.full((1,), -1, jnp.int32)])
    alive_b = input_pos != nxt
    alive = alive_b.astype(jnp.int32)
    cols = jax.lax.iota(jnp.int32, _S)
    onehot = (
        (input_pos[:, None] == cols[None, :]) & alive_b[:, None]
    ).astype(jnp.float32)
    colmask = jnp.sum(onehot, axis=0, keepdims=True)

    mesh = plsc.VectorSubcoreMesh(core_axis_name="c", subcore_axis_name="s")
    vo = pl.kernel(
        _sc_body,
        out_type=jax.ShapeDtypeStruct((_BH, _D, _S), jnp.float32),
        mesh=mesh,
        scratch_types=[
            pltpu.VMEM((_L,), jnp.int32),
            pltpu.VMEM((_L,), jnp.int32),
            pltpu.VMEM((_D, _L), jnp.float32),
            pltpu.VMEM((_D, _CHS), jnp.float32),
        ],
        compiler_params=pltpu.CompilerParams(needs_layout_passes=False),
    )(vct, input_pos, alive, vvt)

    grid = (_BH // _G,)
    cache_spec = pl.BlockSpec((_G, _D, _S), lambda i: (i, 0, 0))
    val_spec = pl.BlockSpec((_G, _D, _L), lambda i: (i, 0, 0))
    oh_spec = pl.BlockSpec((_L, _S), lambda i: (0, 0))
    cm_spec = pl.BlockSpec((1, _S), lambda i: (0, 0))
    ko = pl.pallas_call(
        _tc_body,
        grid=grid,
        in_specs=[cache_spec, val_spec, oh_spec, cm_spec],
        out_specs=cache_spec,
        out_shape=jax.ShapeDtypeStruct((_BH, _D, _S), jnp.float32),
        compiler_params=pltpu.CompilerParams(
            dimension_semantics=("parallel",),
        ),
    )(kct, kvt, onehot, colmask)

    ko = jnp.swapaxes(ko.reshape(_B, _H, _D, _S), 2, 3)
    vo = jnp.swapaxes(vo.reshape(_B, _H, _D, _S), 2, 3)
    return ko, vo
